# baseline (speedup 1.0000x reference)
import functools
import jax
import jax.numpy as jnp
import numpy as np
from jax.experimental import pallas as pl

B, S, D, H, G, DH = 1, 2048, 1024, 16, 4, 64
L, STRIDE, LP, NSEL, W = 32, 16, 64, 8, 512
C = (S - L) // STRIDE + 1
CP = 128
NB = S // LP
HG = H // G
SCALE = 1.0 / np.sqrt(DH)
TS = 256
NQ = S // TS
NEG = -1e30


def _dot(a, b, prec=None):
    return jax.lax.dot_general(a, b, (((1,), (0,)), ((), ())),
                               preferred_element_type=jnp.float32,
                               precision=prec)


def _dot_t(a, b, prec=None):
    return jax.lax.dot_general(a, b, (((1,), (1,)), ((), ())),
                               preferred_element_type=jnp.float32,
                               precision=prec)


def _qkv_kernel(x_ref, w_ref, cos_ref, sin_ref, q_ref, k_ref, v_ref):
    acc = _dot(x_ref[:], w_ref[:])
    cos = cos_ref[:]
    sin = sin_ref[:]
    cos2 = jnp.concatenate([cos, cos], axis=1)
    sin2 = jnp.concatenate([sin, sin], axis=1)
    for h in range(H):
        sl = acc[:, h * DH:(h + 1) * DH]
        a = sl[:, :DH // 2]
        b = sl[:, DH // 2:]
        rot = jnp.concatenate([-b, a], axis=1)
        q_ref[h] = sl * cos2 + rot * sin2
    for g in range(G):
        base = H * DH + g * DH
        sl = acc[:, base:base + DH]
        a = sl[:, :DH // 2]
        b = sl[:, DH // 2:]
        rot = jnp.concatenate([-b, a], axis=1)
        k_ref[g] = sl * cos2 + rot * sin2
        v_ref[g] = acc[:, (H + G) * DH + g * DH:(H + G) * DH + (g + 1) * DH]


def _cmp_kernel(q_ref, k_ref, v_ref, wavg_ref, ovt_ref, wg_ref,
                out_ref, sel_ref):
    HI = jax.lax.Precision.HIGHEST
    kc = _dot(wavg_ref[:], k_ref[0], HI)
    vc = _dot(wavg_ref[:], v_ref[0], HI)
    s_idx = jax.lax.broadcasted_iota(jnp.int32, (S, CP), 0)
    c_idx = jax.lax.broadcasted_iota(jnp.int32, (S, CP), 1)
    valid = (STRIDE * c_idx + L - 1 <= s_idx) & (c_idx < C)
    validf = valid.astype(jnp.float32)
    imp_sum = jnp.zeros((S, CP), jnp.float32)
    for hh in range(HG):
        qh = q_ref[hh]
        sc = _dot_t(qh, kc) * SCALE
        scm = jnp.where(valid, sc, NEG)
        m = jnp.max(scm, axis=1, keepdims=True)
        e = jnp.exp(scm - m) * validf
        l = jnp.sum(e, axis=1, keepdims=True)
        pc = e / jnp.where(l > 0.0, l, 1.0)
        g0 = jax.nn.sigmoid(_dot(qh, wg_ref[:]))[:, 0:1]
        out_ref[hh] = g0 * _dot(pc, vc)
        imp_sum = imp_sum + pc
    imp_t = _dot_t(ovt_ref[:], imp_sum)
    j_idx = jax.lax.broadcasted_iota(jnp.int32, (NB, S), 0)
    s_col = jax.lax.broadcasted_iota(jnp.int32, (NB, S), 1)
    own = (j_idx == s_col // LP).astype(jnp.float32)
    first = (j_idx == 0).astype(jnp.float32)
    imp_t = imp_t + 1e9 * own + 1e9 * first
    a = imp_t[:, None, :]
    bt = imp_t[None, :, :]
    jp = jax.lax.broadcasted_iota(jnp.int32, (NB, NB, 1), 0)
    jj = jax.lax.broadcasted_iota(jnp.int32, (NB, NB, 1), 1)
    cnt = jnp.where((a > bt) | ((a == bt) & (jp < jj)), 1.0, 0.0)
    rank = jnp.sum(cnt, axis=0)
    sel_ref[0] = (rank < NSEL).astype(jnp.float32)


WW = W + TS


def _make_flash(seff, off, aliased):
    def _flash_kernel(q_ref, k_ref, v_ref, sel_ref, e4_ref, wg_ref, *refs):
        out_ref = refs[-1]
        qi = pl.program_id(1) + off
        q4 = q_ref[:].reshape(HG * TS, DH)
        blk_t = sel_ref[0]
        gates = jax.nn.sigmoid(_dot(q4, wg_ref[:]))
        g1 = gates[:, 1:2]
        g2 = gates[:, 2:3]
        s0 = qi * TS

        s_row = s0 + jax.lax.broadcasted_iota(jnp.int32, (TS, seff), 0)
        t_col = jax.lax.broadcasted_iota(jnp.int32, (TS, seff), 1)
        tokf = jax.lax.dot_general(
            blk_t, e4_ref[:], (((0,), (0,)), ((), ())),
            preferred_element_type=jnp.float32)
        bias = (s_row >= t_col).astype(jnp.float32) * tokf * 1e30 - 1e30

        s_row2 = s0 + jax.lax.broadcasted_iota(jnp.int32, (TS, WW), 0)
        t0 = jnp.maximum(qi - (W // TS), 0) * TS
        t_col2 = t0 + jax.lax.broadcasted_iota(jnp.int32, (TS, WW), 1)
        wbias = ((s_row2 >= t_col2) & (t_col2 > s_row2 - W)
                 ).astype(jnp.float32) * 1e30 - 1e30

        sf4 = _dot_t(q4, k_ref[0]) * SCALE
        k_w = k_ref[0, pl.ds(t0, WW), :]
        v_w = v_ref[0, pl.ds(t0, WW), :]
        sw4 = _dot_t(q4, k_w) * SCALE

        for hh in range(HG):
            r = slice(hh * TS, (hh + 1) * TS)
            sl = sf4[r] + bias
            m = jnp.max(sl, axis=1, keepdims=True)
            p = jnp.exp(sl - m)
            l = jnp.sum(p, axis=1, keepdims=True)
            out_sel = _dot(p, v_ref[0]) / l

            sw = sw4[r] + wbias
            mw = jnp.max(sw, axis=1, keepdims=True)
            pw = jnp.exp(sw - mw)
            lw = jnp.sum(pw, axis=1, keepdims=True)
            out_win = _dot(pw, v_w) / lw

            out_ref[hh] = g1[r] * out_sel + g2[r] * out_win
    return _flash_kernel


def _out_kernel(a_ref, b_ref, wo_ref, o_ref):
    comb = jnp.concatenate(
        [a_ref[h] + b_ref[h] for h in range(H)], axis=1)
    o_ref[:] = _dot(comb, wo_ref[:])


@jax.jit
def _run(x, cosS, sinS, WqkvT, WavgC, OvC, E4C, WgP, WoT):
    x2 = x.reshape(S, D)
    q, k, v = pl.pallas_call(
        _qkv_kernel,
        grid=(S // TS,),
        in_specs=[
            pl.BlockSpec((TS, D), lambda i: (i, 0)),
            pl.BlockSpec((D, (H + 2 * G) * DH), lambda i: (0, 0)),
            pl.BlockSpec((TS, DH // 2), lambda i: (i, 0)),
            pl.BlockSpec((TS, DH // 2), lambda i: (i, 0)),
        ],
        out_specs=[
            pl.BlockSpec((H, TS, DH), lambda i: (0, i, 0)),
            pl.BlockSpec((G, TS, DH), lambda i: (0, i, 0)),
            pl.BlockSpec((G, TS, DH), lambda i: (0, i, 0)),
        ],
        out_shape=[
            jax.ShapeDtypeStruct((H, S, DH), jnp.float32),
            jax.ShapeDtypeStruct((G, S, DH), jnp.float32),
            jax.ShapeDtypeStruct((G, S, DH), jnp.float32),
        ],
    )(x2, WqkvT, cosS, sinS)

    out_cmp, blk_sel = pl.pallas_call(
        _cmp_kernel,
        grid=(G,),
        in_specs=[
            pl.BlockSpec((HG, S, DH), lambda g: (g, 0, 0)),
            pl.BlockSpec((1, S, DH), lambda g: (g, 0, 0)),
            pl.BlockSpec((1, S, DH), lambda g: (g, 0, 0)),
            pl.BlockSpec((CP, S), lambda g: (0, 0)),
            pl.BlockSpec((NB, CP), lambda g: (0, 0)),
            pl.BlockSpec((DH, 128), lambda g: (0, 0)),
        ],
        out_specs=[
            pl.BlockSpec((HG, S, DH), lambda g: (g, 0, 0)),
            pl.BlockSpec((1, NB, S), lambda g: (g, 0, 0)),
        ],
        out_shape=[
            jax.ShapeDtypeStruct((H, S, DH), jnp.float32),
            jax.ShapeDtypeStruct((G, NB, S), jnp.float32),
        ],
    )(q, k, v, WavgC, OvC, WgP)

    def flash_call(seff, off, out_prev):
        kw = {} if out_prev is None else {
            "input_output_aliases": {6: 0}}
        args = (q, k, v, blk_sel, E4C, WgP)
        if out_prev is not None:
            args = args + (out_prev,)
            extra = [pl.BlockSpec((HG, TS, DH),
                                  lambda g, qi: (g, qi + off, 0))]
        else:
            extra = []
        return pl.pallas_call(
            _make_flash(seff, off, out_prev is not None),
            grid=(G, NQ // 2),
            in_specs=[
                pl.BlockSpec((HG, TS, DH), lambda g, qi: (g, qi + off, 0)),
                pl.BlockSpec((1, seff, DH), lambda g, qi: (g, 0, 0)),
                pl.BlockSpec((1, seff, DH), lambda g, qi: (g, 0, 0)),
                pl.BlockSpec((1, NB, TS), lambda g, qi: (g, 0, qi + off)),
                pl.BlockSpec((NB, seff), lambda g, qi: (0, 0)),
                pl.BlockSpec((DH, 128), lambda g, qi: (0, 0)),
            ] + extra,
            out_specs=pl.BlockSpec((HG, TS, DH),
                                   lambda g, qi: (g, qi + off, 0)),
            out_shape=jax.ShapeDtypeStruct((H, S, DH), jnp.float32),
            **kw,
        )(*args)

    out_sw = flash_call(S // 2, 0, None)
    out_sw = flash_call(S, NQ // 2, out_sw)

    out = pl.pallas_call(
        _out_kernel,
        grid=(S // TS,),
        in_specs=[
            pl.BlockSpec((H, TS, DH), lambda i: (0, i, 0)),
            pl.BlockSpec((H, TS, DH), lambda i: (0, i, 0)),
            pl.BlockSpec((H * DH, D), lambda i: (0, 0)),
        ],
        out_specs=pl.BlockSpec((TS, D), lambda i: (i, 0)),
        out_shape=jax.ShapeDtypeStruct((S, D), jnp.float32),
    )(out_cmp, out_sw, WoT)
    return out.reshape(B, S, D)


def kernel(x, start_pos, freqs_cis, Wq, Wk, Wv, Wo, Wg):
    perm = np.concatenate([np.arange(0, DH, 2), np.arange(1, DH, 2)])
    Wq_p = Wq.reshape(H, DH, D)[:, perm].reshape(H * DH, D)
    Wk_p = Wk.reshape(G, DH, D)[:, perm].reshape(G * DH, D)
    WqkvT = jnp.concatenate([Wq_p, Wk_p, Wv], axis=0).T
    WgP = jnp.zeros((DH, 128), jnp.float32).at[:, :3].set(Wg[perm])
    cosS = freqs_cis[..., 0]
    sinS = freqs_cis[..., 1]
    c = np.arange(CP)
    t = np.arange(S)
    wavg = ((t[None, :] >= STRIDE * c[:, None])
            & (t[None, :] < STRIDE * c[:, None] + L)
            & (c[:, None] < C)).astype(np.float32) / L
    j = np.arange(NB)
    ov = ((STRIDE * c[None, :] <= LP * j[:, None] + LP - 1)
          & (STRIDE * c[None, :] + L - 1 >= LP * j[:, None])
          & (c[None, :] < C)).astype(np.float32)
    e4 = (t[None, :] // LP == j[:, None]).astype(np.float32)
    return _run(x, cosS, sinS, WqkvT,
                jnp.asarray(wavg), jnp.asarray(ov), jnp.asarray(e4),
                WgP, jnp.asarray(Wo.T))


# speedup vs baseline: 1.1167x; 1.1167x over previous
import functools
import jax
import jax.numpy as jnp
import numpy as np
from jax.experimental import pallas as pl

B, S, D, H, G, DH = 1, 2048, 1024, 16, 4, 64
L, STRIDE, LP, NSEL, W = 32, 16, 64, 8, 512
C = (S - L) // STRIDE + 1
CP = 128
NB = S // LP
HG = H // G
SCALE = 1.0 / np.sqrt(DH)
TS = 256
NQ = S // TS
NEG = -1e30


def _dot(a, b, prec=None):
    return jax.lax.dot_general(a, b, (((1,), (0,)), ((), ())),
                               preferred_element_type=jnp.float32,
                               precision=prec)


def _dot_t(a, b, prec=None):
    return jax.lax.dot_general(a, b, (((1,), (1,)), ((), ())),
                               preferred_element_type=jnp.float32,
                               precision=prec)


def _qkv_kernel(x_ref, w_ref, cos_ref, sin_ref, q_ref, k_ref, v_ref):
    acc = _dot(x_ref[:], w_ref[:])
    cos = cos_ref[:]
    sin = sin_ref[:]
    cos2 = jnp.concatenate([cos, cos], axis=1)
    sin2 = jnp.concatenate([sin, sin], axis=1)
    for h in range(H):
        sl = acc[:, h * DH:(h + 1) * DH]
        a = sl[:, :DH // 2]
        b = sl[:, DH // 2:]
        rot = jnp.concatenate([-b, a], axis=1)
        q_ref[h] = sl * cos2 + rot * sin2
    for g in range(G):
        base = H * DH + g * DH
        sl = acc[:, base:base + DH]
        a = sl[:, :DH // 2]
        b = sl[:, DH // 2:]
        rot = jnp.concatenate([-b, a], axis=1)
        k_ref[g] = sl * cos2 + rot * sin2
        v_ref[g] = acc[:, (H + G) * DH + g * DH:(H + G) * DH + (g + 1) * DH]


def _cmp_kernel(q_ref, k_ref, v_ref, wavg_ref, ovt_ref, wg_ref,
                out_ref, sel_ref):
    HI = jax.lax.Precision.HIGHEST
    kc = _dot(wavg_ref[:], k_ref[0], HI)
    vc = _dot(wavg_ref[:], v_ref[0], HI)
    s_idx = jax.lax.broadcasted_iota(jnp.int32, (S, CP), 0)
    c_idx = jax.lax.broadcasted_iota(jnp.int32, (S, CP), 1)
    valid = (STRIDE * c_idx + L - 1 <= s_idx) & (c_idx < C)
    validf = valid.astype(jnp.float32)
    imp_sum = jnp.zeros((S, CP), jnp.float32)
    for hh in range(HG):
        qh = q_ref[hh]
        sc = _dot_t(qh, kc) * SCALE
        scm = jnp.where(valid, sc, NEG)
        m = jnp.max(scm, axis=1, keepdims=True)
        e = jnp.exp(scm - m) * validf
        l = jnp.sum(e, axis=1, keepdims=True)
        pc = e / jnp.where(l > 0.0, l, 1.0)
        g0 = jax.nn.sigmoid(_dot(qh, wg_ref[:]))[:, 0:1]
        out_ref[hh] = g0 * _dot(pc, vc)
        imp_sum = imp_sum + pc
    imp_t = _dot_t(ovt_ref[:], imp_sum)
    j_idx = jax.lax.broadcasted_iota(jnp.int32, (NB, S), 0)
    s_col = jax.lax.broadcasted_iota(jnp.int32, (NB, S), 1)
    own = (j_idx == s_col // LP).astype(jnp.float32)
    first = (j_idx == 0).astype(jnp.float32)
    imp_t = imp_t + 1e9 * own + 1e9 * first
    a = imp_t[:, None, :]
    bt = imp_t[None, :, :]
    jp = jax.lax.broadcasted_iota(jnp.int32, (NB, NB, 1), 0)
    jj = jax.lax.broadcasted_iota(jnp.int32, (NB, NB, 1), 1)
    cnt = jnp.where((a > bt) | ((a == bt) & (jp < jj)), 1.0, 0.0)
    rank = jnp.sum(cnt, axis=0)
    sel_ref[0] = (rank < NSEL).astype(jnp.float32)


WW = W + TS


def _make_flash(seff, off, aliased):
    def _flash_kernel(q_ref, k_ref, v_ref, sel_ref, e4_ref, wg_ref, *refs):
        out_ref = refs[-1]
        qi = pl.program_id(1) + off
        q4 = q_ref[:].reshape(HG * TS, DH)
        blk_t = sel_ref[0]
        gates = jax.nn.sigmoid(_dot(q4, wg_ref[:]))
        g1 = gates[:, 1:2]
        g2 = gates[:, 2:3]
        s0 = qi * TS

        s_row = s0 + jax.lax.broadcasted_iota(jnp.int32, (TS, seff), 0)
        t_col = jax.lax.broadcasted_iota(jnp.int32, (TS, seff), 1)
        tokf = jax.lax.dot_general(
            blk_t, e4_ref[:], (((0,), (0,)), ((), ())),
            preferred_element_type=jnp.float32)
        bias = (s_row >= t_col).astype(jnp.float32) * tokf * 1e30 - 1e30

        s_row2 = s0 + jax.lax.broadcasted_iota(jnp.int32, (TS, WW), 0)
        t0 = jnp.maximum(qi - (W // TS), 0) * TS
        t_col2 = t0 + jax.lax.broadcasted_iota(jnp.int32, (TS, WW), 1)
        wbias = ((s_row2 >= t_col2) & (t_col2 > s_row2 - W)
                 ).astype(jnp.float32) * 1e30 - 1e30

        q4s = q4 * (SCALE * 1.4426950408889634)
        sf4 = _dot_t(q4s, k_ref[0])
        k_w = k_ref[0, pl.ds(t0, WW), :]
        v_w = v_ref[0, pl.ds(t0, WW), :]
        sw4 = _dot_t(q4s, k_w)

        for hh in range(HG):
            r = slice(hh * TS, (hh + 1) * TS)
            p = jnp.exp2(sf4[r] + bias)
            l = jnp.sum(p, axis=1, keepdims=True)
            out_sel = _dot(p, v_ref[0]) / l

            pw = jnp.exp2(sw4[r] + wbias)
            lw = jnp.sum(pw, axis=1, keepdims=True)
            out_win = _dot(pw, v_w) / lw

            out_ref[hh] = g1[r] * out_sel + g2[r] * out_win
    return _flash_kernel


def _out_kernel(a_ref, b_ref, wo_ref, o_ref):
    comb = jnp.concatenate(
        [a_ref[h] + b_ref[h] for h in range(H)], axis=1)
    o_ref[:] = _dot(comb, wo_ref[:])


@jax.jit
def _run(x, cosS, sinS, WqkvT, WavgC, OvC, E4C, WgP, WoT):
    x2 = x.reshape(S, D)
    q, k, v = pl.pallas_call(
        _qkv_kernel,
        grid=(S // TS,),
        in_specs=[
            pl.BlockSpec((TS, D), lambda i: (i, 0)),
            pl.BlockSpec((D, (H + 2 * G) * DH), lambda i: (0, 0)),
            pl.BlockSpec((TS, DH // 2), lambda i: (i, 0)),
            pl.BlockSpec((TS, DH // 2), lambda i: (i, 0)),
        ],
        out_specs=[
            pl.BlockSpec((H, TS, DH), lambda i: (0, i, 0)),
            pl.BlockSpec((G, TS, DH), lambda i: (0, i, 0)),
            pl.BlockSpec((G, TS, DH), lambda i: (0, i, 0)),
        ],
        out_shape=[
            jax.ShapeDtypeStruct((H, S, DH), jnp.float32),
            jax.ShapeDtypeStruct((G, S, DH), jnp.float32),
            jax.ShapeDtypeStruct((G, S, DH), jnp.float32),
        ],
    )(x2, WqkvT, cosS, sinS)

    out_cmp, blk_sel = pl.pallas_call(
        _cmp_kernel,
        grid=(G,),
        in_specs=[
            pl.BlockSpec((HG, S, DH), lambda g: (g, 0, 0)),
            pl.BlockSpec((1, S, DH), lambda g: (g, 0, 0)),
            pl.BlockSpec((1, S, DH), lambda g: (g, 0, 0)),
            pl.BlockSpec((CP, S), lambda g: (0, 0)),
            pl.BlockSpec((NB, CP), lambda g: (0, 0)),
            pl.BlockSpec((DH, 128), lambda g: (0, 0)),
        ],
        out_specs=[
            pl.BlockSpec((HG, S, DH), lambda g: (g, 0, 0)),
            pl.BlockSpec((1, NB, S), lambda g: (g, 0, 0)),
        ],
        out_shape=[
            jax.ShapeDtypeStruct((H, S, DH), jnp.float32),
            jax.ShapeDtypeStruct((G, NB, S), jnp.float32),
        ],
    )(q, k, v, WavgC, OvC, WgP)

    def flash_call(seff, off, out_prev):
        kw = {} if out_prev is None else {
            "input_output_aliases": {6: 0}}
        args = (q, k, v, blk_sel, E4C, WgP)
        if out_prev is not None:
            args = args + (out_prev,)
            extra = [pl.BlockSpec((HG, TS, DH),
                                  lambda g, qi: (g, qi + off, 0))]
        else:
            extra = []
        return pl.pallas_call(
            _make_flash(seff, off, out_prev is not None),
            grid=(G, NQ // 2),
            in_specs=[
                pl.BlockSpec((HG, TS, DH), lambda g, qi: (g, qi + off, 0)),
                pl.BlockSpec((1, seff, DH), lambda g, qi: (g, 0, 0)),
                pl.BlockSpec((1, seff, DH), lambda g, qi: (g, 0, 0)),
                pl.BlockSpec((1, NB, TS), lambda g, qi: (g, 0, qi + off)),
                pl.BlockSpec((NB, seff), lambda g, qi: (0, 0)),
                pl.BlockSpec((DH, 128), lambda g, qi: (0, 0)),
            ] + extra,
            out_specs=pl.BlockSpec((HG, TS, DH),
                                   lambda g, qi: (g, qi + off, 0)),
            out_shape=jax.ShapeDtypeStruct((H, S, DH), jnp.float32),
            **kw,
        )(*args)

    out_sw = flash_call(S // 2, 0, None)
    out_sw = flash_call(S, NQ // 2, out_sw)

    out = pl.pallas_call(
        _out_kernel,
        grid=(S // TS,),
        in_specs=[
            pl.BlockSpec((H, TS, DH), lambda i: (0, i, 0)),
            pl.BlockSpec((H, TS, DH), lambda i: (0, i, 0)),
            pl.BlockSpec((H * DH, D), lambda i: (0, 0)),
        ],
        out_specs=pl.BlockSpec((TS, D), lambda i: (i, 0)),
        out_shape=jax.ShapeDtypeStruct((S, D), jnp.float32),
    )(out_cmp, out_sw, WoT)
    return out.reshape(B, S, D)


def kernel(x, start_pos, freqs_cis, Wq, Wk, Wv, Wo, Wg):
    perm = np.concatenate([np.arange(0, DH, 2), np.arange(1, DH, 2)])
    Wq_p = Wq.reshape(H, DH, D)[:, perm].reshape(H * DH, D)
    Wk_p = Wk.reshape(G, DH, D)[:, perm].reshape(G * DH, D)
    WqkvT = jnp.concatenate([Wq_p, Wk_p, Wv], axis=0).T
    WgP = jnp.zeros((DH, 128), jnp.float32).at[:, :3].set(Wg[perm])
    cosS = freqs_cis[..., 0]
    sinS = freqs_cis[..., 1]
    c = np.arange(CP)
    t = np.arange(S)
    wavg = ((t[None, :] >= STRIDE * c[:, None])
            & (t[None, :] < STRIDE * c[:, None] + L)
            & (c[:, None] < C)).astype(np.float32) / L
    j = np.arange(NB)
    ov = ((STRIDE * c[None, :] <= LP * j[:, None] + LP - 1)
          & (STRIDE * c[None, :] + L - 1 >= LP * j[:, None])
          & (c[None, :] < C)).astype(np.float32)
    e4 = (t[None, :] // LP == j[:, None]).astype(np.float32)
    return _run(x, cosS, sinS, WqkvT,
                jnp.asarray(wavg), jnp.asarray(ov), jnp.asarray(e4),
                WgP, jnp.asarray(Wo.T))


# speedup vs baseline: 1.2952x; 1.1599x over previous
import functools
import jax
import jax.numpy as jnp
import numpy as np
from jax.experimental import pallas as pl

B, S, D, H, G, DH = 1, 2048, 1024, 16, 4, 64
L, STRIDE, LP, NSEL, W = 32, 16, 64, 8, 512
C = (S - L) // STRIDE + 1
CP = 128
NB = S // LP
HG = H // G
SCALE = 1.0 / np.sqrt(DH)
TS = 256
NQ = S // TS
NEG = -1e30


def _dot(a, b, prec=None):
    return jax.lax.dot_general(a, b, (((1,), (0,)), ((), ())),
                               preferred_element_type=jnp.float32,
                               precision=prec)


def _dot_t(a, b, prec=None):
    return jax.lax.dot_general(a, b, (((1,), (1,)), ((), ())),
                               preferred_element_type=jnp.float32,
                               precision=prec)


def _qkv_kernel(x_ref, w_ref, cos_ref, sin_ref, wavg_ref,
                q_ref, k_ref, v_ref, kc_ref, vc_ref):
    i = pl.program_id(0)
    acc = _dot(x_ref[:], w_ref[:])
    cos = cos_ref[:]
    sin = sin_ref[:]
    cos2 = jnp.concatenate([cos, cos], axis=1)
    sin2 = jnp.concatenate([sin, sin], axis=1)
    for h in range(H):
        sl = acc[:, h * DH:(h + 1) * DH]
        a = sl[:, :DH // 2]
        b = sl[:, DH // 2:]
        rot = jnp.concatenate([-b, a], axis=1)
        q_ref[h] = (sl * cos2 + rot * sin2).astype(jnp.bfloat16)
    HI = jax.lax.Precision.HIGHEST
    wv = wavg_ref[:]

    @pl.when(i == 0)
    def _():
        kc_ref[...] = jnp.zeros_like(kc_ref)
        vc_ref[...] = jnp.zeros_like(vc_ref)

    for g in range(G):
        base = H * DH + g * DH
        sl = acc[:, base:base + DH]
        a = sl[:, :DH // 2]
        b = sl[:, DH // 2:]
        rot = jnp.concatenate([-b, a], axis=1)
        kr = sl * cos2 + rot * sin2
        vr = acc[:, (H + G) * DH + g * DH:(H + G) * DH + (g + 1) * DH]
        k_ref[g] = kr.astype(jnp.bfloat16)
        v_ref[g] = vr.astype(jnp.bfloat16)
        kc_ref[g] += _dot(wv, kr, HI)
        vc_ref[g] += _dot(wv, vr, HI)


def _cmp_kernel(q_ref, kc_ref, vc_ref, ovt_ref, wg_ref,
                out_ref, sel_ref):
    kc = kc_ref[0]
    vc = vc_ref[0]
    s_idx = jax.lax.broadcasted_iota(jnp.int32, (S, CP), 0)
    c_idx = jax.lax.broadcasted_iota(jnp.int32, (S, CP), 1)
    valid = (STRIDE * c_idx + L - 1 <= s_idx) & (c_idx < C)
    validf = valid.astype(jnp.float32)
    imp_sum = jnp.zeros((S, CP), jnp.float32)
    for hh in range(HG):
        qh = q_ref[hh].astype(jnp.float32)
        sc = _dot_t(qh, kc) * SCALE
        scm = jnp.where(valid, sc, NEG)
        m = jnp.max(scm, axis=1, keepdims=True)
        e = jnp.exp(scm - m) * validf
        l = jnp.sum(e, axis=1, keepdims=True)
        pc = e / jnp.where(l > 0.0, l, 1.0)
        g0 = jax.nn.sigmoid(_dot(qh, wg_ref[:]))[:, 0:1]
        out_ref[hh] = (g0 * _dot(pc, vc)).astype(jnp.bfloat16)
        imp_sum = imp_sum + pc
    imp_t = _dot_t(ovt_ref[:], imp_sum)
    j_idx = jax.lax.broadcasted_iota(jnp.int32, (NB, S), 0)
    s_col = jax.lax.broadcasted_iota(jnp.int32, (NB, S), 1)
    own = (j_idx == s_col // LP).astype(jnp.float32)
    first = (j_idx == 0).astype(jnp.float32)
    imp_t = imp_t + 1e9 * own + 1e9 * first
    a = imp_t[:, None, :]
    bt = imp_t[None, :, :]
    jp = jax.lax.broadcasted_iota(jnp.int32, (NB, NB, 1), 0)
    jj = jax.lax.broadcasted_iota(jnp.int32, (NB, NB, 1), 1)
    cnt = jnp.where((a > bt) | ((a == bt) & (jp < jj)), 1.0, 0.0)
    rank = jnp.sum(cnt, axis=0)
    sel_ref[0] = (rank < NSEL).astype(jnp.float32)


WW = W + TS


def _make_flash(seff, off, aliased):
    def _flash_kernel(q_ref, k_ref, v_ref, sel_ref, e4_ref, wg_ref, *refs):
        out_ref = refs[-1]
        qi = pl.program_id(1) + off
        q4 = q_ref[:].reshape(HG * TS, DH).astype(jnp.float32)
        blk_t = sel_ref[0]
        gates = jax.nn.sigmoid(_dot(q4, wg_ref[:]))
        g1 = gates[:, 1:2]
        g2 = gates[:, 2:3]
        s0 = qi * TS

        s_row = s0 + jax.lax.broadcasted_iota(jnp.int32, (TS, seff), 0)
        t_col = jax.lax.broadcasted_iota(jnp.int32, (TS, seff), 1)
        tokf = jax.lax.dot_general(
            blk_t, e4_ref[:], (((0,), (0,)), ((), ())),
            preferred_element_type=jnp.float32)
        bias = (s_row >= t_col).astype(jnp.float32) * tokf * 1e30 - 1e30

        s_row2 = s0 + jax.lax.broadcasted_iota(jnp.int32, (TS, WW), 0)
        t0 = jnp.maximum(qi - (W // TS), 0) * TS
        t_col2 = t0 + jax.lax.broadcasted_iota(jnp.int32, (TS, WW), 1)
        wbias = ((s_row2 >= t_col2) & (t_col2 > s_row2 - W)
                 ).astype(jnp.float32) * 1e30 - 1e30

        q4s = q4 * (SCALE * 1.4426950408889634)
        k_full = k_ref[0].astype(jnp.float32)
        v_full = v_ref[0].astype(jnp.float32)
        sf4 = _dot_t(q4s, k_full)
        k_w = k_ref[0, pl.ds(t0, WW), :].astype(jnp.float32)
        v_w = v_ref[0, pl.ds(t0, WW), :].astype(jnp.float32)
        sw4 = _dot_t(q4s, k_w)

        for hh in range(HG):
            r = slice(hh * TS, (hh + 1) * TS)
            p = jnp.exp2(sf4[r] + bias)
            l = jnp.sum(p, axis=1, keepdims=True)
            out_sel = _dot(p, v_full) / l

            pw = jnp.exp2(sw4[r] + wbias)
            lw = jnp.sum(pw, axis=1, keepdims=True)
            out_win = _dot(pw, v_w) / lw

            out_ref[hh] = (g1[r] * out_sel
                           + g2[r] * out_win).astype(jnp.bfloat16)
    return _flash_kernel


def _out_kernel(a_ref, b_ref, wo_ref, o_ref):
    comb = jnp.concatenate(
        [a_ref[h].astype(jnp.float32) + b_ref[h].astype(jnp.float32)
         for h in range(H)], axis=1)
    o_ref[:] = _dot(comb, wo_ref[:])


@jax.jit
def _run(x, cosS, sinS, WqkvT, WavgC, OvC, E4C, WgP, WoT):
    x2 = x.reshape(S, D)
    q, k, v, kc, vc = pl.pallas_call(
        _qkv_kernel,
        grid=(S // TS,),
        in_specs=[
            pl.BlockSpec((TS, D), lambda i: (i, 0)),
            pl.BlockSpec((D, (H + 2 * G) * DH), lambda i: (0, 0)),
            pl.BlockSpec((TS, DH // 2), lambda i: (i, 0)),
            pl.BlockSpec((TS, DH // 2), lambda i: (i, 0)),
            pl.BlockSpec((CP, TS), lambda i: (0, i)),
        ],
        out_specs=[
            pl.BlockSpec((H, TS, DH), lambda i: (0, i, 0)),
            pl.BlockSpec((G, TS, DH), lambda i: (0, i, 0)),
            pl.BlockSpec((G, TS, DH), lambda i: (0, i, 0)),
            pl.BlockSpec((G, CP, DH), lambda i: (0, 0, 0)),
            pl.BlockSpec((G, CP, DH), lambda i: (0, 0, 0)),
        ],
        out_shape=[
            jax.ShapeDtypeStruct((H, S, DH), jnp.bfloat16),
            jax.ShapeDtypeStruct((G, S, DH), jnp.bfloat16),
            jax.ShapeDtypeStruct((G, S, DH), jnp.bfloat16),
            jax.ShapeDtypeStruct((G, CP, DH), jnp.float32),
            jax.ShapeDtypeStruct((G, CP, DH), jnp.float32),
        ],
    )(x2, WqkvT, cosS, sinS, WavgC)

    out_cmp, blk_sel = pl.pallas_call(
        _cmp_kernel,
        grid=(G,),
        in_specs=[
            pl.BlockSpec((HG, S, DH), lambda g: (g, 0, 0)),
            pl.BlockSpec((1, CP, DH), lambda g: (g, 0, 0)),
            pl.BlockSpec((1, CP, DH), lambda g: (g, 0, 0)),
            pl.BlockSpec((NB, CP), lambda g: (0, 0)),
            pl.BlockSpec((DH, 128), lambda g: (0, 0)),
        ],
        out_specs=[
            pl.BlockSpec((HG, S, DH), lambda g: (g, 0, 0)),
            pl.BlockSpec((1, NB, S), lambda g: (g, 0, 0)),
        ],
        out_shape=[
            jax.ShapeDtypeStruct((H, S, DH), jnp.bfloat16),
            jax.ShapeDtypeStruct((G, NB, S), jnp.float32),
        ],
    )(q, kc, vc, OvC, WgP)

    def flash_call(seff, off, out_prev):
        kw = {} if out_prev is None else {
            "input_output_aliases": {6: 0}}
        args = (q, k, v, blk_sel, E4C, WgP)
        if out_prev is not None:
            args = args + (out_prev,)
            extra = [pl.BlockSpec((HG, TS, DH),
                                  lambda g, qi: (g, qi + off, 0))]
        else:
            extra = []
        return pl.pallas_call(
            _make_flash(seff, off, out_prev is not None),
            grid=(G, NQ // 2),
            in_specs=[
                pl.BlockSpec((HG, TS, DH), lambda g, qi: (g, qi + off, 0)),
                pl.BlockSpec((1, seff, DH), lambda g, qi: (g, 0, 0)),
                pl.BlockSpec((1, seff, DH), lambda g, qi: (g, 0, 0)),
                pl.BlockSpec((1, NB, TS), lambda g, qi: (g, 0, qi + off)),
                pl.BlockSpec((NB, seff), lambda g, qi: (0, 0)),
                pl.BlockSpec((DH, 128), lambda g, qi: (0, 0)),
            ] + extra,
            out_specs=pl.BlockSpec((HG, TS, DH),
                                   lambda g, qi: (g, qi + off, 0)),
            out_shape=jax.ShapeDtypeStruct((H, S, DH), jnp.bfloat16),
            **kw,
        )(*args)

    out_sw = flash_call(S // 2, 0, None)
    out_sw = flash_call(S, NQ // 2, out_sw)

    out = pl.pallas_call(
        _out_kernel,
        grid=(S // TS,),
        in_specs=[
            pl.BlockSpec((H, TS, DH), lambda i: (0, i, 0)),
            pl.BlockSpec((H, TS, DH), lambda i: (0, i, 0)),
            pl.BlockSpec((H * DH, D), lambda i: (0, 0)),
        ],
        out_specs=pl.BlockSpec((TS, D), lambda i: (i, 0)),
        out_shape=jax.ShapeDtypeStruct((S, D), jnp.float32),
    )(out_cmp, out_sw, WoT)
    return out.reshape(B, S, D)


def kernel(x, start_pos, freqs_cis, Wq, Wk, Wv, Wo, Wg):
    perm = np.concatenate([np.arange(0, DH, 2), np.arange(1, DH, 2)])
    Wq_p = Wq.reshape(H, DH, D)[:, perm].reshape(H * DH, D)
    Wk_p = Wk.reshape(G, DH, D)[:, perm].reshape(G * DH, D)
    WqkvT = jnp.concatenate([Wq_p, Wk_p, Wv], axis=0).T
    WgP = jnp.zeros((DH, 128), jnp.float32).at[:, :3].set(Wg[perm])
    cosS = freqs_cis[..., 0]
    sinS = freqs_cis[..., 1]
    c = np.arange(CP)
    t = np.arange(S)
    wavg = ((t[None, :] >= STRIDE * c[:, None])
            & (t[None, :] < STRIDE * c[:, None] + L)
            & (c[:, None] < C)).astype(np.float32) / L
    j = np.arange(NB)
    ov = ((STRIDE * c[None, :] <= LP * j[:, None] + LP - 1)
          & (STRIDE * c[None, :] + L - 1 >= LP * j[:, None])
          & (c[None, :] < C)).astype(np.float32)
    e4 = (t[None, :] // LP == j[:, None]).astype(np.float32)
    return _run(x, cosS, sinS, WqkvT,
                jnp.asarray(wavg), jnp.asarray(ov), jnp.asarray(e4),
                WgP, jnp.asarray(Wo.T))
